# baseline (speedup 1.0000x reference)
import functools

import jax
import jax.numpy as jnp
from jax.experimental import pallas as pl
from jax.experimental.pallas import tpu as pltpu

N = 384
NT = 2 * N
D = 128
BI = 32
BLOCKS_PER_SET = N // BI
REG = 0.001
INVREG = 1.0 / REG
SINK_ITERS = 100
RPAD = 392
CPAD = 512
NEG = -1e30

_HI = jax.lax.Precision.HIGHEST


def _mm(a, b):
    return jax.lax.dot_general(a, b, (((1,), (0,)), ((), ())),
                               precision=_HI, preferred_element_type=jnp.float32)


def _enc_qkv_kernel(p_ref, d_ref, f1w_ref, f1b_ref, f2w_ref, f2b_ref,
                    w1_ref, b1_ref, w2_ref, b2_ref, w3_ref, b3_ref,
                    q_ref, k_ref, v_ref, kt_ref, vt_ref):
    p = p_ref[...]
    f1w = f1w_ref[...]
    h = p[:, 0:1] * f1w[0:1, :] + p[:, 1:2] * f1w[1:2, :] + f1b_ref[...]
    h = jnp.maximum(h, 0.0)
    x = jnp.maximum(_mm(h, f2w_ref[...]) + f2b_ref[...], 0.0) + d_ref[...]
    q_ref[...] = _mm(x, w1_ref[...]) + b1_ref[...]
    k = _mm(x, w2_ref[...]) + b2_ref[...]
    v = _mm(x, w3_ref[...]) + b3_ref[...]
    k_ref[...] = k
    v_ref[...] = v
    kt_ref[...] = k.T
    vt_ref[...] = v.T


def _enc_qkv(p, d, f1w, f1b, f2w, f2b, w1, b1, w2, b2, w3, b3):
    out = jax.ShapeDtypeStruct((NT, D), jnp.float32)
    out_t = jax.ShapeDtypeStruct((D, NT), jnp.float32)
    return pl.pallas_call(
        _enc_qkv_kernel,
        out_shape=(out, out, out, out_t, out_t),
    )(p, d, f1w, f1b, f2w, f2b, w1, b1, w2, b2, w3, b3)


def _qkv_kernel(x_ref, w1_ref, b1_ref, w2_ref, b2_ref, w3_ref, b3_ref,
                q_ref, k_ref, v_ref, kt_ref, vt_ref):
    x = x_ref[...]
    q_ref[...] = _mm(x, w1_ref[...]) + b1_ref[...]
    k = _mm(x, w2_ref[...]) + b2_ref[...]
    v = _mm(x, w3_ref[...]) + b3_ref[...]
    k_ref[...] = k
    v_ref[...] = v
    kt_ref[...] = k.T
    vt_ref[...] = v.T


def _qkv(x, w1, b1, w2, b2, w3, b3):
    out = jax.ShapeDtypeStruct((NT, D), jnp.float32)
    out_t = jax.ShapeDtypeStruct((D, NT), jnp.float32)
    return pl.pallas_call(
        _qkv_kernel,
        out_shape=(out, out, out, out_t, out_t),
    )(x, w1, b1, w2, b2, w3, b3)


def _att_body_t(q, kt, vt):
    t = q[:, :, None] * kt[None, :, :]
    mx = jnp.max(t, axis=1, keepdims=True)
    e = jnp.exp(t - mx)
    z = jnp.sum(e, axis=1, keepdims=True)
    w = vt[None, :, :] * (1.0 / z)
    return jnp.sum(e * w, axis=2)


def _diag_term(q, k_loc, v_loc):
    td = q * k_loc
    mxd = jnp.max(td, axis=1, keepdims=True)
    ed = jnp.exp(td - mxd)
    zd = jnp.sum(ed, axis=1, keepdims=True)
    return ed * (v_loc / zd)


def _att_intra_kernel(residual, q_ref, kt_ref, vt_ref, kd_ref, vd_ref,
                      x_ref, o_ref):
    q = q_ref[...]
    msg = _att_body_t(q, kt_ref[...], vt_ref[...])
    msg = msg - _diag_term(q, kd_ref[...], vd_ref[...])
    if residual:
        msg = msg + x_ref[...]
    o_ref[...] = msg


def _att_intra(q, kt, vt, k, v, x, residual):
    nblk = 2 * BLOCKS_PER_SET
    return pl.pallas_call(
        functools.partial(_att_intra_kernel, residual),
        grid=(nblk,),
        in_specs=[
            pl.BlockSpec((BI, D), lambda i: (i, 0)),
            pl.BlockSpec((D, N), lambda i: (0, i // BLOCKS_PER_SET)),
            pl.BlockSpec((D, N), lambda i: (0, i // BLOCKS_PER_SET)),
            pl.BlockSpec((BI, D), lambda i: (i, 0)),
            pl.BlockSpec((BI, D), lambda i: (i, 0)),
            pl.BlockSpec((BI, D), lambda i: (i, 0)),
        ],
        out_specs=pl.BlockSpec((BI, D), lambda i: (i, 0)),
        out_shape=jax.ShapeDtypeStruct((NT, D), jnp.float32),
        compiler_params=pltpu.CompilerParams(
            dimension_semantics=("arbitrary",)),
    )(q, kt, vt, k, v, x)


def _att_cross_kernel(q_ref, kt_ref, vt_ref, x_ref, o_ref):
    msg = _att_body_t(q_ref[...], kt_ref[...], vt_ref[...])
    o_ref[...] = msg + x_ref[...]


def _att_cross(q, kt, vt, x):
    return pl.pallas_call(
        _att_cross_kernel,
        grid=(BLOCKS_PER_SET,),
        in_specs=[
            pl.BlockSpec((BI, D), lambda i: (i + BLOCKS_PER_SET, 0)),
            pl.BlockSpec((D, N), lambda i: (0, 0)),
            pl.BlockSpec((D, N), lambda i: (0, 0)),
            pl.BlockSpec((BI, D), lambda i: (i + BLOCKS_PER_SET, 0)),
        ],
        out_specs=pl.BlockSpec((BI, D), lambda i: (i, 0)),
        out_shape=jax.ShapeDtypeStruct((N, D), jnp.float32),
        compiler_params=pltpu.CompilerParams(
            dimension_semantics=("arbitrary",)),
    )(q, kt, vt, x)


def _final_kernel(h_ref, fw_ref, fb_ref, dust_ref, m_ref, o_ref):
    h = jnp.maximum(_mm(h_ref[...], fw_ref[...]) + fb_ref[...], 0.0)
    h = h / jnp.sqrt(jnp.sum(h * h, axis=1, keepdims=True))
    v1 = h[:N, :]
    v2 = h[N:, :]
    costs = jax.lax.dot_general(v1, v2, (((1,), (1,)), ((), ())),
                                precision=_HI,
                                preferred_element_type=jnp.float32)
    w = dust_ref[0, 0]
    cpad = jnp.pad(costs, ((0, RPAD - N), (0, CPAD - N)))
    ri = jax.lax.broadcasted_iota(jnp.int32, (RPAD, CPAD), 0)
    ci = jax.lax.broadcasted_iota(jnp.int32, (RPAD, CPAD), 1)
    interior = (ri < N) & (ci < N)
    boundary = (ri <= N) & (ci <= N) & ~interior
    m_mat = jnp.where(interior, 1.0 - cpad, jnp.where(boundary, 1.0 - w, 0.0))

    rv = jax.lax.broadcasted_iota(jnp.int32, (RPAD, 1), 0)
    cv = jax.lax.broadcasted_iota(jnp.int32, (1, CPAD), 1)
    row_valid = rv <= N
    col_valid = cv <= N
    loga = jnp.where(rv == N, jnp.log(float(N)), 0.0)
    logb = jnp.where(cv == N, jnp.log(float(N)), 0.0)

    def body(_, fg):
        f, g = fg
        xr = jnp.where(col_valid, (g - m_mat) * INVREG, NEG)
        mr = jnp.max(xr, axis=1, keepdims=True)
        lser = mr + jnp.log(jnp.sum(jnp.exp(xr - mr), axis=1, keepdims=True))
        f = jnp.where(row_valid, REG * (loga - lser), 0.0)
        xc = jnp.where(row_valid, (f - m_mat) * INVREG, NEG)
        mc = jnp.max(xc, axis=0, keepdims=True)
        lsec = mc + jnp.log(jnp.sum(jnp.exp(xc - mc), axis=0, keepdims=True))
        g = jnp.where(col_valid, REG * (logb - lsec), 0.0)
        return f, g

    f0 = jnp.zeros((RPAD, 1), jnp.float32)
    g0 = jnp.zeros((1, CPAD), jnp.float32)
    f, g = jax.lax.fori_loop(0, SINK_ITERS, body, (f0, g0))

    sol = jnp.where((rv < N) & (cv < N),
                    jnp.exp((f + g - m_mat) * INVREG), 0.0)
    r = m_ref[:, 0:1]
    c = m_ref[:, 1:2]
    rr = jax.lax.broadcasted_iota(jnp.int32, (256, RPAD), 1)
    r_onehot = (rr == r).astype(jnp.float32)
    picked = _mm(r_onehot, sol)
    cc = jax.lax.broadcasted_iota(jnp.int32, (256, CPAD), 1)
    c_onehot = (cc == c).astype(jnp.float32)
    vals = jnp.sum(picked * c_onehot, axis=1, keepdims=True)
    loss = jnp.sum(-jnp.log(vals + 0.001)) * (1.0 / 256.0)
    o_ref[...] = loss.reshape(1, 1)


def _final(h, fw, fb, dust, matches):
    return pl.pallas_call(
        _final_kernel,
        out_shape=jax.ShapeDtypeStruct((1, 1), jnp.float32),
    )(h, fw, fb, dust, matches)


def kernel(p1, d1, p2, d2, matches, params):
    p = jnp.concatenate([p1[0], p2[0]], axis=0)
    d = jnp.concatenate([d1[0], d2[0]], axis=0)
    pr = params
    b = lambda name: pr[name].reshape(1, -1)

    q, k, v, kt, vt = _enc_qkv(p, d,
                               pr['fc1_w'], b('fc1_b'), pr['fc2_w'], b('fc2_b'),
                               pr['mp1_W1'], b('mp1_b1'), pr['mp1_W2'],
                               b('mp1_b2'), pr['mp1_W3'], b('mp1_b3'))
    h1 = _att_intra(q, kt, vt, k, v, q, residual=False)

    q, k, v, kt, vt = _qkv(h1, pr['mp2_W1'], b('mp2_b1'), pr['mp2_W2'],
                           b('mp2_b2'), pr['mp2_W3'], b('mp2_b3'))
    h2b = _att_cross(q, kt, vt, h1)
    h2 = jnp.concatenate([h1[:N], h2b], axis=0)

    q, k, v, kt, vt = _qkv(h2, pr['mp3_W1'], b('mp3_b1'), pr['mp3_W2'],
                           b('mp3_b2'), pr['mp3_W3'], b('mp3_b3'))
    h3 = _att_intra(q, kt, vt, k, v, h2, residual=True)

    q, k, v, kt, vt = _qkv(h3, pr['mp4_W1'], b('mp4_b1'), pr['mp4_W2'],
                           b('mp4_b2'), pr['mp4_W3'], b('mp4_b3'))
    h4b = _att_cross(q, kt, vt, h3)
    h4 = jnp.concatenate([h3[:N], h4b], axis=0)

    loss = _final(h4, pr['fc3_w'], b('fc3_b'),
                  pr['dustbin'].reshape(1, 1), matches)
    return loss.reshape(())


# speedup vs baseline: 1.1378x; 1.1378x over previous
import jax
import jax.numpy as jnp
from jax.experimental import pallas as pl

N = 384
NT = 2 * N
D = 128
BI = 32
NBLK = N // BI
REG = 0.001
INVREG = 1.0 / REG
SINK_ITERS = 100
RPAD = 392
CPAD = 512
NEG = -1e30
LOG2E = 1.4426950408889634

_HI = jax.lax.Precision.HIGHEST


def _mm(a, b):
    return jax.lax.dot_general(a, b, (((1,), (0,)), ((), ())),
                               precision=_HI, preferred_element_type=jnp.float32)


def _att_msgs(qs, k, v, kd, vd):
    t = qs[:, None, :] * k[None, :, :]
    mx = jnp.max(t, axis=2, keepdims=True)
    e = jnp.exp2(t - mx)
    z = jnp.sum(e, axis=2, keepdims=True)
    msg = jnp.sum(e * (1.0 / z) * v[None, :, :], axis=1)
    if kd is not None:
        td = qs * kd
        mxd = jnp.max(td, axis=1, keepdims=True)
        ed = jnp.exp2(td - mxd)
        zd = jnp.sum(ed, axis=1, keepdims=True)
        msg = msg - ed * (vd / zd)
    return msg


def _layer(x, w1, b1, w2, b2, w3, b3, q_s, k_s, v_s, o_s, cross):
    q_s[...] = (_mm(x, w1) + b1) * LOG2E
    k_s[...] = _mm(x, w2) + b2
    v_s[...] = _mm(x, w3) + b3

    if cross:
        k1 = k_s[0:N, :]
        v1 = v_s[0:N, :]

        def blk(i, carry):
            r0 = N + i * BI
            msg = _att_msgs(q_s[pl.ds(r0, BI), :], k1, v1, None, None)
            o_s[pl.ds(r0, BI), :] = msg
            return carry

        jax.lax.fori_loop(0, NBLK, blk, 0)
    else:
        for s in (0, 1):
            ks = k_s[pl.ds(s * N, N), :]
            vs = v_s[pl.ds(s * N, N), :]

            def blk(i, carry):
                r0 = s * N + i * BI
                qs = q_s[pl.ds(r0, BI), :]
                msg = _att_msgs(qs, ks, vs,
                                k_s[pl.ds(r0, BI), :], v_s[pl.ds(r0, BI), :])
                o_s[pl.ds(r0, BI), :] = msg
                return carry

            jax.lax.fori_loop(0, NBLK, blk, 0)


def _mega_kernel(p_ref, d_ref, m_ref,
                 f1w_ref, f1b_ref, f2w_ref, f2b_ref,
                 lw_refs,
                 f3w_ref, f3b_ref, dust_ref,
                 o_ref, q_s, k_s, v_s, o_s):
    p = p_ref[...]
    f1w = f1w_ref[...]
    h = p[:, 0:1] * f1w[0:1, :] + p[:, 1:2] * f1w[1:2, :] + f1b_ref[...]
    h = jnp.maximum(h, 0.0)
    x = jnp.maximum(_mm(h, f2w_ref[...]) + f2b_ref[...], 0.0) + d_ref[...]

    for li, cross in enumerate((False, True, False, True)):
        w1, b1, w2, b2, w3, b3 = (r[...] for r in lw_refs[li])
        _layer(x, w1, b1, w2, b2, w3, b3, q_s, k_s, v_s, o_s, cross)
        msgs = o_s[...]
        if li == 0:
            x = msgs
        elif cross:
            x = jnp.concatenate([x[0:N, :], x[N:, :] + msgs[N:, :]], axis=0)
        else:
            x = x + msgs

    h5 = jnp.maximum(_mm(x, f3w_ref[...]) + f3b_ref[...], 0.0)
    h5 = h5 / jnp.sqrt(jnp.sum(h5 * h5, axis=1, keepdims=True))
    costs = jax.lax.dot_general(h5[0:N, :], h5[N:, :], (((1,), (1,)), ((), ())),
                                precision=_HI,
                                preferred_element_type=jnp.float32)
    w = dust_ref[0, 0]
    cp = jnp.pad(costs, ((0, RPAD - N), (0, CPAD - N)))
    ri = jax.lax.broadcasted_iota(jnp.int32, (RPAD, CPAD), 0)
    ci = jax.lax.broadcasted_iota(jnp.int32, (RPAD, CPAD), 1)
    interior = (ri < N) & (ci < N)
    boundary = (ri <= N) & (ci <= N) & ~interior
    m_mat = jnp.where(interior, 1.0 - cp, jnp.where(boundary, 1.0 - w, 0.0))

    rv = jax.lax.broadcasted_iota(jnp.int32, (RPAD, 1), 0)
    cv = jax.lax.broadcasted_iota(jnp.int32, (1, CPAD), 1)
    row_valid = rv <= N
    col_valid = cv <= N
    loga = jnp.where(rv == N, jnp.log(float(N)), 0.0)
    logb = jnp.where(cv == N, jnp.log(float(N)), 0.0)

    def body(_, fg):
        f, g = fg
        xr = jnp.where(col_valid, (g - m_mat) * INVREG, NEG)
        mr = jnp.max(xr, axis=1, keepdims=True)
        lser = mr + jnp.log(jnp.sum(jnp.exp(xr - mr), axis=1, keepdims=True))
        f = jnp.where(row_valid, REG * (loga - lser), 0.0)
        xc = jnp.where(row_valid, (f - m_mat) * INVREG, NEG)
        mc = jnp.max(xc, axis=0, keepdims=True)
        lsec = mc + jnp.log(jnp.sum(jnp.exp(xc - mc), axis=0, keepdims=True))
        g = jnp.where(col_valid, REG * (logb - lsec), 0.0)
        return f, g

    f0 = jnp.zeros((RPAD, 1), jnp.float32)
    g0 = jnp.zeros((1, CPAD), jnp.float32)
    f, g = jax.lax.fori_loop(0, SINK_ITERS, body, (f0, g0))

    sol = jnp.where((rv < N) & (cv < N),
                    jnp.exp((f + g - m_mat) * INVREG), 0.0)
    r = m_ref[:, 0:1]
    c = m_ref[:, 1:2]
    rr = jax.lax.broadcasted_iota(jnp.int32, (256, RPAD), 1)
    r_onehot = (rr == r).astype(jnp.float32)
    picked = _mm(r_onehot, sol)
    cc = jax.lax.broadcasted_iota(jnp.int32, (256, CPAD), 1)
    c_onehot = (cc == c).astype(jnp.float32)
    vals = jnp.sum(picked * c_onehot, axis=1, keepdims=True)
    loss = jnp.sum(-jnp.log(vals + 0.001)) * (1.0 / 256.0)
    o_ref[...] = loss.reshape(1, 1)


def _mega_entry(*refs):
    fixed = refs[:7]
    lw = [refs[7 + 6 * i:7 + 6 * (i + 1)] for i in range(4)]
    tail = refs[31:34]
    o_ref = refs[34]
    scratch = refs[35:]
    _mega_kernel(*fixed, lw, *tail, o_ref, *scratch)


def kernel(p1, d1, p2, d2, matches, params):
    from jax.experimental.pallas import tpu as pltpu

    p = jnp.concatenate([p1[0], p2[0]], axis=0)
    d = jnp.concatenate([d1[0], d2[0]], axis=0)
    pr = params
    b = lambda name: pr[name].reshape(1, -1)

    args = [p, d, matches,
            pr['fc1_w'], b('fc1_b'), pr['fc2_w'], b('fc2_b')]
    for l in range(1, 5):
        args += [pr['mp%d_W1' % l], b('mp%d_b1' % l),
                 pr['mp%d_W2' % l], b('mp%d_b2' % l),
                 pr['mp%d_W3' % l], b('mp%d_b3' % l)]
    args += [pr['fc3_w'], b('fc3_b'), pr['dustbin'].reshape(1, 1)]

    loss = pl.pallas_call(
        _mega_entry,
        out_shape=jax.ShapeDtypeStruct((1, 1), jnp.float32),
        scratch_shapes=[pltpu.VMEM((NT, D), jnp.float32)] * 4,
    )(*args)
    return loss.reshape(())
